# baseline (speedup 1.0000x reference)
import functools

import jax
import jax.numpy as jnp
from jax.experimental import pallas as pl
from jax.experimental.pallas import tpu as pltpu

_N_NODES = 102
_IN = 2
_HID = 512
_EMB = 256
_FCHUNK = 64


def _fused_kernel(x_ref, rc_ref, wg_ref, bg_ref, wp_ref, bp_ref, o_ref,
                  me_ref, mo_ref):
    fc = x_ref.shape[1]

    @pl.when((pl.program_id(0) == 0) & (pl.program_id(1) == 0))
    def _build_adjacency():
        rc = rc_ref[...]
        row = rc[0:1, :]
        col = rc[1:2, :]
        e_tot = rc.shape[1]
        iota_n = jax.lax.broadcasted_iota(jnp.int32, (_N_NODES, e_tot), 0)
        cmask = (iota_n == col).astype(jnp.float32)
        rmask = (iota_n == row).astype(jnp.float32)
        deg = jnp.sum(cmask, axis=1, keepdims=True)
        dinv = jnp.where(deg > 0, jax.lax.rsqrt(deg), 0.0)
        dinv_col = jnp.sum(cmask * dinv, axis=0, keepdims=True)
        dinv_row = jnp.sum(rmask * dinv, axis=0, keepdims=True)
        norm = dinv_row * dinv_col
        iota_k = jax.lax.broadcasted_iota(jnp.int32, (2 * _N_NODES, e_tot), 0)
        pe = (iota_k == 2 * row).astype(jnp.float32)
        po = (iota_k == 2 * row + 1).astype(jnp.float32)
        q = cmask * norm
        dims = (((1,), (1,)), ((), ()))
        me_ref[...] = jax.lax.dot_general(pe, q, dims,
                                          preferred_element_type=jnp.float32)
        mo_ref[...] = jax.lax.dot_general(po, q, dims,
                                          preferred_element_type=jnp.float32)

    x = x_ref[0]
    cdims = (((0,), (1,)), ((), ()))
    yet = jax.lax.dot_general(me_ref[...], x, cdims,
                              preferred_element_type=jnp.float32)
    yot = jax.lax.dot_general(mo_ref[...], x, cdims,
                              preferred_element_type=jnp.float32)
    wg = wg_ref[...]
    h = (yet[:, :, None] * wg[0][None, None, :]
         + yot[:, :, None] * wg[1][None, None, :]
         + bg_ref[...][None])
    h = jnp.maximum(h, 0.0).reshape(_N_NODES * fc, _HID)
    o2 = (jnp.dot(h, wp_ref[...], preferred_element_type=jnp.float32)
          + bp_ref[...])
    o_ref[0] = o2.reshape(_N_NODES, fc, _EMB)


@functools.partial(jax.jit, static_argnames=())
def kernel(x, edge_index, W_gcn, b_gcn, W_proj, b_proj):
    B, n, _ = x.shape

    loops = jnp.arange(_N_NODES, dtype=edge_index.dtype)
    rc = jnp.concatenate(
        [edge_index, jnp.stack([loops, loops], axis=0)], axis=1)

    grid = (B, n // _FCHUNK)

    out_nm = pl.pallas_call(
        _fused_kernel,
        grid=grid,
        in_specs=[
            pl.BlockSpec((1, _FCHUNK, _IN * _N_NODES), lambda b, j: (b, j, 0)),
            pl.BlockSpec(rc.shape, lambda b, j: (0, 0)),
            pl.BlockSpec((_IN, _HID), lambda b, j: (0, 0)),
            pl.BlockSpec((1, _HID), lambda b, j: (0, 0)),
            pl.BlockSpec((_HID, _EMB), lambda b, j: (0, 0)),
            pl.BlockSpec((1, _EMB), lambda b, j: (0, 0)),
        ],
        out_specs=pl.BlockSpec((1, _N_NODES, _FCHUNK, _EMB),
                               lambda b, j: (b, 0, j, 0)),
        out_shape=jax.ShapeDtypeStruct((B, _N_NODES, n, _EMB), jnp.float32),
        scratch_shapes=[
            pltpu.VMEM((_IN * _N_NODES, _N_NODES), jnp.float32),
            pltpu.VMEM((_IN * _N_NODES, _N_NODES), jnp.float32),
        ],
    )(x, rc, W_gcn, b_gcn.reshape(1, _HID), W_proj, b_proj.reshape(1, _EMB))

    return jnp.transpose(out_nm, (0, 2, 1, 3))


# speedup vs baseline: 1.0180x; 1.0180x over previous
import functools

import jax
import jax.numpy as jnp
from jax.experimental import pallas as pl
from jax.experimental.pallas import tpu as pltpu

_N_NODES = 102
_IN = 2
_HID = 512
_EMB = 256
_FCHUNK = 128


def _fused_kernel(x_ref, rc_ref, wg_ref, bg_ref, wp_ref, bp_ref, o_ref,
                  me_ref, mo_ref):
    fc = x_ref.shape[1]

    @pl.when((pl.program_id(0) == 0) & (pl.program_id(1) == 0))
    def _build_adjacency():
        rc = rc_ref[...]
        row = rc[0:1, :]
        col = rc[1:2, :]
        e_tot = rc.shape[1]
        iota_n = jax.lax.broadcasted_iota(jnp.int32, (_N_NODES, e_tot), 0)
        cmask = (iota_n == col).astype(jnp.float32)
        rmask = (iota_n == row).astype(jnp.float32)
        deg = jnp.sum(cmask, axis=1, keepdims=True)
        dinv = jnp.where(deg > 0, jax.lax.rsqrt(deg), 0.0)
        dinv_col = jnp.sum(cmask * dinv, axis=0, keepdims=True)
        dinv_row = jnp.sum(rmask * dinv, axis=0, keepdims=True)
        norm = dinv_row * dinv_col
        iota_k = jax.lax.broadcasted_iota(jnp.int32, (2 * _N_NODES, e_tot), 0)
        pe = (iota_k == 2 * row).astype(jnp.float32)
        po = (iota_k == 2 * row + 1).astype(jnp.float32)
        q = cmask * norm
        dims = (((1,), (1,)), ((), ()))
        me_ref[...] = jax.lax.dot_general(pe, q, dims,
                                          preferred_element_type=jnp.float32)
        mo_ref[...] = jax.lax.dot_general(po, q, dims,
                                          preferred_element_type=jnp.float32)

    x = x_ref[0]
    cdims = (((0,), (1,)), ((), ()))
    yet = jax.lax.dot_general(me_ref[...], x, cdims,
                              preferred_element_type=jnp.float32)
    yot = jax.lax.dot_general(mo_ref[...], x, cdims,
                              preferred_element_type=jnp.float32)
    wg = wg_ref[...]
    h = (yet[:, :, None] * wg[0][None, None, :]
         + yot[:, :, None] * wg[1][None, None, :]
         + bg_ref[...][None])
    h = jnp.maximum(h, 0.0).reshape(_N_NODES * fc, _HID)
    o2 = (jnp.dot(h, wp_ref[...], preferred_element_type=jnp.float32)
          + bp_ref[...])
    o_ref[0] = o2.reshape(_N_NODES, fc, _EMB)


@functools.partial(jax.jit, static_argnames=())
def kernel(x, edge_index, W_gcn, b_gcn, W_proj, b_proj):
    B, n, _ = x.shape

    loops = jnp.arange(_N_NODES, dtype=edge_index.dtype)
    rc = jnp.concatenate(
        [edge_index, jnp.stack([loops, loops], axis=0)], axis=1)

    grid = (B, n // _FCHUNK)

    out_nm = pl.pallas_call(
        _fused_kernel,
        grid=grid,
        in_specs=[
            pl.BlockSpec((1, _FCHUNK, _IN * _N_NODES), lambda b, j: (b, j, 0)),
            pl.BlockSpec(rc.shape, lambda b, j: (0, 0)),
            pl.BlockSpec((_IN, _HID), lambda b, j: (0, 0)),
            pl.BlockSpec((1, _HID), lambda b, j: (0, 0)),
            pl.BlockSpec((_HID, _EMB), lambda b, j: (0, 0)),
            pl.BlockSpec((1, _EMB), lambda b, j: (0, 0)),
        ],
        out_specs=pl.BlockSpec((1, _N_NODES, _FCHUNK, _EMB),
                               lambda b, j: (b, 0, j, 0)),
        out_shape=jax.ShapeDtypeStruct((B, _N_NODES, n, _EMB), jnp.float32),
        scratch_shapes=[
            pltpu.VMEM((_IN * _N_NODES, _N_NODES), jnp.float32),
            pltpu.VMEM((_IN * _N_NODES, _N_NODES), jnp.float32),
        ],
    )(x, rc, W_gcn, b_gcn.reshape(1, _HID), W_proj, b_proj.reshape(1, _EMB))

    return jnp.transpose(out_nm, (0, 2, 1, 3))


# speedup vs baseline: 1.0227x; 1.0047x over previous
import functools

import jax
import jax.numpy as jnp
from jax.experimental import pallas as pl
from jax.experimental.pallas import tpu as pltpu

_N_NODES = 102
_IN = 2
_HID = 512
_EMB = 256
_FCHUNK = 128


def _fused_kernel(x_ref, rc_ref, wg_ref, bg_ref, wp_ref, bp_ref, o_ref,
                  me_ref, mo_ref):
    fc = x_ref.shape[1]

    @pl.when((pl.program_id(0) == 0) & (pl.program_id(1) == 0))
    def _build_adjacency():
        rc = rc_ref[...]
        row = rc[0:1, :]
        col = rc[1:2, :]
        e_tot = rc.shape[1]
        iota_n = jax.lax.broadcasted_iota(jnp.int32, (_N_NODES, e_tot), 0)
        cmask = (iota_n == col).astype(jnp.float32)
        rmask = (iota_n == row).astype(jnp.float32)
        deg = jnp.sum(cmask, axis=1, keepdims=True)
        dinv = jnp.where(deg > 0, jax.lax.rsqrt(deg), 0.0)
        dinv_col = jnp.sum(cmask * dinv, axis=0, keepdims=True)
        dinv_row = jnp.sum(rmask * dinv, axis=0, keepdims=True)
        norm = dinv_row * dinv_col
        iota_k = jax.lax.broadcasted_iota(jnp.int32, (2 * _N_NODES, e_tot), 0)
        pe = (iota_k == 2 * row).astype(jnp.float32)
        po = (iota_k == 2 * row + 1).astype(jnp.float32)
        q = cmask * norm
        dims = (((1,), (1,)), ((), ()))
        me_ref[...] = jax.lax.dot_general(pe, q, dims,
                                          preferred_element_type=jnp.float32)
        mo_ref[...] = jax.lax.dot_general(po, q, dims,
                                          preferred_element_type=jnp.float32)

    x = x_ref[0]
    cdims = (((0,), (1,)), ((), ()))
    yet = jax.lax.dot_general(me_ref[...], x, cdims,
                              preferred_element_type=jnp.float32)
    yot = jax.lax.dot_general(mo_ref[...], x, cdims,
                              preferred_element_type=jnp.float32)
    wg = wg_ref[...]
    h = (yet[:, :, None] * wg[0][None, None, :]
         + yot[:, :, None] * wg[1][None, None, :]
         + bg_ref[...][None])
    h = jnp.maximum(h, 0.0).reshape(_N_NODES * fc, _HID)
    o2 = (jnp.dot(h, wp_ref[...], preferred_element_type=jnp.float32)
          + bp_ref[...])
    o_ref[0] = o2.reshape(_N_NODES, fc, _EMB)


@functools.partial(jax.jit, static_argnames=())
def kernel(x, edge_index, W_gcn, b_gcn, W_proj, b_proj):
    B, n, _ = x.shape

    loops = jnp.arange(_N_NODES, dtype=edge_index.dtype)
    rc = jnp.concatenate(
        [edge_index, jnp.stack([loops, loops], axis=0)], axis=1)

    grid = (B, n // _FCHUNK)

    out_nm = pl.pallas_call(
        _fused_kernel,
        grid=grid,
        in_specs=[
            pl.BlockSpec((1, _FCHUNK, _IN * _N_NODES), lambda b, j: (b, j, 0)),
            pl.BlockSpec(rc.shape, lambda b, j: (0, 0)),
            pl.BlockSpec((_IN, _HID), lambda b, j: (0, 0)),
            pl.BlockSpec((1, _HID), lambda b, j: (0, 0)),
            pl.BlockSpec((_HID, _EMB), lambda b, j: (0, 0)),
            pl.BlockSpec((1, _EMB), lambda b, j: (0, 0)),
        ],
        out_specs=pl.BlockSpec((1, _N_NODES, _FCHUNK, _EMB),
                               lambda b, j: (b, 0, j, 0)),
        out_shape=jax.ShapeDtypeStruct((B, _N_NODES, n, _EMB), jnp.float32),
        scratch_shapes=[
            pltpu.VMEM((_IN * _N_NODES, _N_NODES), jnp.float32),
            pltpu.VMEM((_IN * _N_NODES, _N_NODES), jnp.float32),
        ],
        compiler_params=pltpu.CompilerParams(
            vmem_limit_bytes=100 * 1024 * 1024,
        ),
    )(x, rc, W_gcn, b_gcn.reshape(1, _HID), W_proj, b_proj.reshape(1, _EMB))

    return jnp.transpose(out_nm, (0, 2, 1, 3))
